# Initial kernel scaffold; baseline (speedup 1.0000x reference)
import jax
import jax.numpy as jnp
from jax.experimental import pallas as pl


def kernel(x, first_W1, first_b1, first_g, first_beta, first_W2, first_b2, fake_W1, fake_b1, fake_g, fake_beta, fake_W2, fake_b2, real_W1, real_b1, real_g, real_beta, real_W2, real_b2):
    raise NotImplementedError("write your pallas kernel here")


# speedup vs baseline: 1.2136x; 1.2136x over previous
import jax
import jax.numpy as jnp
from jax.experimental import pallas as pl
from jax.experimental.pallas import tpu as pltpu

N = 8192
D = 1024
TM = 512


def _ln_relu(h, g, beta):
    mu = jnp.mean(h, axis=-1, keepdims=True)
    var = jnp.mean((h - mu) * (h - mu), axis=-1, keepdims=True)
    h = (h - mu) / jnp.sqrt(var + 1e-5) * g + beta
    return jnp.maximum(h, 0.0)


def _dense_body(x_ref, w1_ref, b1_ref, g_ref, beta_ref, w2_ref, b2_ref,
                route_ref, out_ref):
    x = x_ref[...]
    logits = []
    for e in range(3):
        h = jnp.dot(x, w1_ref[e], preferred_element_type=jnp.float32)
        h = h + b1_ref[e]
        h = _ln_relu(h, g_ref[e], beta_ref[e])
        logits.append(jnp.dot(h, w2_ref[e], preferred_element_type=jnp.float32)
                      + b2_ref[e])
    l_first, l_fake, l_real = logits
    route = (l_first[:, 1:2] > l_first[:, 0:1]).astype(jnp.int32)
    route_ref[...] = route
    out_ref[...] = jnp.where(route == 0, l_fake, l_real)


def kernel(x,
           first_W1, first_b1, first_g, first_beta, first_W2, first_b2,
           fake_W1, fake_b1, fake_g, fake_beta, fake_W2, fake_b2,
           real_W1, real_b1, real_g, real_beta, real_W2, real_b2):
    W1s = jnp.stack([first_W1, fake_W1, real_W1])
    b1s = jnp.stack([first_b1, fake_b1, real_b1])[:, None, :]
    gs = jnp.stack([first_g, fake_g, real_g])[:, None, :]
    betas = jnp.stack([first_beta, fake_beta, real_beta])[:, None, :]
    W2s = jnp.stack([first_W2, fake_W2, real_W2])
    b2s = jnp.stack([first_b2, fake_b2, real_b2])[:, None, :]

    full = lambda shape: pl.BlockSpec(shape, lambda i: (0,) * len(shape))
    route, out = pl.pallas_call(
        _dense_body,
        grid=(N // TM,),
        in_specs=[
            pl.BlockSpec((TM, D), lambda i: (i, 0)),
            full((3, D, D)),
            full((3, 1, D)),
            full((3, 1, D)),
            full((3, 1, D)),
            full((3, D, 2)),
            full((3, 1, 2)),
        ],
        out_specs=[
            pl.BlockSpec((TM, 1), lambda i: (i, 0)),
            pl.BlockSpec((TM, 2), lambda i: (i, 0)),
        ],
        out_shape=[
            jax.ShapeDtypeStruct((N, 1), jnp.int32),
            jax.ShapeDtypeStruct((N, 2), jnp.float32),
        ],
        compiler_params=pltpu.CompilerParams(
            dimension_semantics=("parallel",)),
    )(x, W1s, b1s, gs, betas, W2s, b2s)
    return route.reshape(N), out
